# Initial kernel scaffold; baseline (speedup 1.0000x reference)
import jax
import jax.numpy as jnp
from jax.experimental import pallas as pl


def kernel(x):
    raise NotImplementedError("write your pallas kernel here")


# speedup vs baseline: 20.5773x; 20.5773x over previous
import functools

import jax
import jax.numpy as jnp
from jax import lax
from jax.experimental import pallas as pl

_N_NEIGHBORS = 5
_BIG = 3.0e38


def _aug_block_kernel(x_ref, xt_ref, ranksel_ref, alpha_ref, out_ref, *, blk_r, n):
    i = pl.program_id(0)
    r0 = pl.multiple_of(i * blk_r, blk_r)
    xb = x_ref[pl.ds(r0, blk_r), :]

    dots = lax.dot_general(
        xb, xt_ref[:],
        (((1,), (0,)), ((), ())),
        precision=lax.Precision.DEFAULT,
        preferred_element_type=jnp.float32,
    )
    sq_cols = jnp.sum(xt_ref[:] * xt_ref[:], axis=0, keepdims=True)
    sq_rows = jnp.sum(xb * xb, axis=1, keepdims=True)
    d2 = sq_rows + sq_cols - 2.0 * dots

    cols_i = lax.broadcasted_iota(jnp.int32, (blk_r, n), 1)
    rows_i = i * blk_r + lax.broadcasted_iota(jnp.int32, (blk_r, n), 0)
    d2 = jnp.where(cols_i == rows_i, _BIG, d2)
    cols_f = cols_i.astype(jnp.float32)

    nb_f = jnp.zeros((blk_r, 1), jnp.float32)
    for r in range(_N_NEIGHBORS):
        m = jnp.min(d2, axis=1, keepdims=True)
        idxm = jnp.min(jnp.where(d2 == m, cols_f, _BIG), axis=1, keepdims=True)
        nb_f = nb_f + ranksel_ref[:, r:r + 1] * idxm
        if r + 1 < _N_NEIGHBORS:
            d2 = jnp.where(cols_f == idxm, _BIG, d2)

    onehot = (cols_f == nb_f).astype(jnp.float32)
    neighbors = lax.dot_general(
        onehot, x_ref[:],
        (((1,), (0,)), ((), ())),
        precision=lax.Precision.HIGHEST,
        preferred_element_type=jnp.float32,
    )
    a = alpha_ref[:, 0:1]
    out_ref[:, :] = xb + a * (neighbors - xb)


def kernel(x):
    n, d = x.shape
    blk_r = 256
    nb_blocks = n // blk_r

    key = jax.random.key(1)
    k1, k2 = jax.random.split(key)
    choice = jax.random.randint(k1, (1, n), 0, _N_NEIGHBORS)[0]
    alpha = jax.random.uniform(k2, (1, n, 1), dtype=x.dtype)[0]

    ranksel = (choice[:, None] == jnp.arange(8)[None, :]).astype(jnp.float32)
    alpha8 = jnp.broadcast_to(alpha, (n, 8))

    xt = x.T

    aug = pl.pallas_call(
        functools.partial(_aug_block_kernel, blk_r=blk_r, n=n),
        grid=(nb_blocks,),
        in_specs=[
            pl.BlockSpec((n, d), lambda i: (0, 0)),
            pl.BlockSpec((d, n), lambda i: (0, 0)),
            pl.BlockSpec((blk_r, 8), lambda i: (i, 0)),
            pl.BlockSpec((blk_r, 8), lambda i: (i, 0)),
        ],
        out_specs=pl.BlockSpec((blk_r, d), lambda i: (i, 0)),
        out_shape=jax.ShapeDtypeStruct((n, d), jnp.float32),
    )(x, xt, ranksel, alpha8)

    return jnp.concatenate([x, aug], axis=0)


# speedup vs baseline: 26.5169x; 1.2886x over previous
import functools

import jax
import jax.numpy as jnp
from jax import lax
from jax.experimental import pallas as pl
from jax.experimental.pallas import tpu as pltpu
from jax.experimental.pallas import tpu_sc as plsc

_N_NEIGHBORS = 5
_BIG = 3.0e38

_NC = 2
_NS = 16
_LANES = 16


def _knn_idx_kernel(x_ref, xt_ref, ranksel_ref, out_ref, *, blk_r, n):
    i = pl.program_id(0)
    r0 = pl.multiple_of(i * blk_r, blk_r)
    xb = x_ref[pl.ds(r0, blk_r), :]

    dots = lax.dot_general(
        xb, xt_ref[:],
        (((1,), (0,)), ((), ())),
        precision=lax.Precision.DEFAULT,
        preferred_element_type=jnp.float32,
    )
    sq_cols = jnp.sum(xt_ref[:] * xt_ref[:], axis=0, keepdims=True)
    sq_rows = jnp.sum(xb * xb, axis=1, keepdims=True)
    d2 = sq_rows + sq_cols - 2.0 * dots

    cols_i = lax.broadcasted_iota(jnp.int32, (blk_r, n), 1)
    rows_i = i * blk_r + lax.broadcasted_iota(jnp.int32, (blk_r, n), 0)
    d2 = jnp.where(cols_i == rows_i, _BIG, d2)
    cols_f = cols_i.astype(jnp.float32)

    nb_f = jnp.zeros((blk_r, 1), jnp.float32)
    for r in range(_N_NEIGHBORS):
        m = jnp.min(d2, axis=1, keepdims=True)
        idxm = jnp.min(jnp.where(d2 == m, cols_f, _BIG), axis=1, keepdims=True)
        nb_f = nb_f + ranksel_ref[:, r:r + 1] * idxm
        if r + 1 < _N_NEIGHBORS:
            d2 = jnp.where(cols_f == idxm, _BIG, d2)

    out_ref[:, :] = jnp.broadcast_to(nb_f, (blk_r, 8))


def _sc_augment(x_hbm, idx_hbm, al_hbm, out_hbm, idx_v, nbr_v, mine_v, al_v, sem,
                *, n, d, rows_w):
    wid = lax.axis_index("s") * _NC + lax.axis_index("c")
    base = wid * rows_w

    pltpu.sync_copy(idx_hbm.at[pl.ds(base, rows_w)], idx_v)
    gather = pltpu.async_copy(x_hbm.at[idx_v], nbr_v, sem)
    pltpu.sync_copy(x_hbm.at[pl.ds(base, rows_w)], mine_v)
    pltpu.sync_copy(mine_v, out_hbm.at[pl.ds(base, rows_w)])
    pltpu.sync_copy(al_hbm.at[pl.ds(base, rows_w)], al_v)
    gather.wait()

    nchunk = d // _LANES

    def row_body(r, _):
        for c in range(nchunk):
            s = pl.ds(c * _LANES, _LANES)
            mine = mine_v[r, s]
            a = al_v[r, s]
            nbr_v[r, s] = mine + a * (nbr_v[r, s] - mine)
        return 0

    lax.fori_loop(0, rows_w, row_body, 0)
    pltpu.sync_copy(nbr_v, out_hbm.at[pl.ds(n + base, rows_w)])


def kernel(x):
    n, d = x.shape
    blk_r = 256
    nb_blocks = n // blk_r
    nw = _NC * _NS
    rows_w = n // nw

    key = jax.random.key(1)
    k1, k2 = jax.random.split(key)
    choice = jax.random.randint(k1, (1, n), 0, _N_NEIGHBORS)[0]
    alpha = jax.random.uniform(k2, (1, n, 1), dtype=x.dtype)[0]

    ranksel = (choice[:, None] == jnp.arange(8)[None, :]).astype(jnp.float32)
    alpha_full = jnp.broadcast_to(alpha, (n, d))

    xt = x.T

    nb_idx_f = pl.pallas_call(
        functools.partial(_knn_idx_kernel, blk_r=blk_r, n=n),
        grid=(nb_blocks,),
        in_specs=[
            pl.BlockSpec((n, d), lambda i: (0, 0)),
            pl.BlockSpec((d, n), lambda i: (0, 0)),
            pl.BlockSpec((blk_r, 8), lambda i: (i, 0)),
        ],
        out_specs=pl.BlockSpec((blk_r, 8), lambda i: (i, 0)),
        out_shape=jax.ShapeDtypeStruct((n, 8), jnp.float32),
    )(x, xt, ranksel)

    nb_idx = nb_idx_f[:, 0].astype(jnp.int32)

    sc = functools.partial(
        pl.kernel,
        out_type=jax.ShapeDtypeStruct((2 * n, d), jnp.float32),
        mesh=plsc.VectorSubcoreMesh(core_axis_name="c", subcore_axis_name="s"),
        scratch_types=[
            pltpu.VMEM((rows_w,), jnp.int32),
            pltpu.VMEM((rows_w, d), jnp.float32),
            pltpu.VMEM((rows_w, d), jnp.float32),
            pltpu.VMEM((rows_w, d), jnp.float32),
            pltpu.SemaphoreType.DMA,
        ],
    )(functools.partial(_sc_augment, n=n, d=d, rows_w=rows_w))

    return sc(x, nb_idx, alpha_full)


# speedup vs baseline: 32.4578x; 1.2240x over previous
import functools

import jax
import jax.numpy as jnp
from jax import lax
from jax.experimental import pallas as pl
from jax.experimental.pallas import tpu as pltpu
from jax.experimental.pallas import tpu_sc as plsc

_N_NEIGHBORS = 5
_BIG = 3.0e38

_NC = 2
_NS = 16
_LANES = 16


def _knn_idx_kernel(x_ref, xt_ref, ranksel_ref, out_ref, *, blk_r, n):
    i = pl.program_id(0)
    r0 = pl.multiple_of(i * blk_r, blk_r)
    xb = x_ref[pl.ds(r0, blk_r), :]

    dots = lax.dot_general(
        xb, xt_ref[:],
        (((1,), (0,)), ((), ())),
        precision=lax.Precision.DEFAULT,
        preferred_element_type=jnp.float32,
    )
    sq_cols = jnp.sum(xt_ref[:] * xt_ref[:], axis=0, keepdims=True)
    sq_rows = jnp.sum(xb * xb, axis=1, keepdims=True)
    d2 = sq_rows + sq_cols - 2.0 * dots

    cols_i = lax.broadcasted_iota(jnp.int32, (blk_r, n), 1)
    rows_i = i * blk_r + lax.broadcasted_iota(jnp.int32, (blk_r, n), 0)
    d2 = jnp.where(cols_i == rows_i, _BIG, d2)
    cols_f = cols_i.astype(jnp.float32)

    vstar = jnp.zeros((blk_r, 1), jnp.float32)
    m = jnp.min(d2, axis=1, keepdims=True)
    vstar = vstar + ranksel_ref[:, 0:1] * m
    for r in range(1, _N_NEIGHBORS):
        m = jnp.min(jnp.where(d2 > m, d2, _BIG), axis=1, keepdims=True)
        vstar = vstar + ranksel_ref[:, r:r + 1] * m
    nb_f = jnp.min(jnp.where(d2 == vstar, cols_f, _BIG), axis=1, keepdims=True)

    out_ref[:, :] = jnp.broadcast_to(nb_f, (blk_r, 8))


def _sc_augment(x_hbm, idx_hbm, al_hbm, out_hbm, idx_v, nbr_v, mine_v, al_v, sem,
                *, n, d, rows_w):
    wid = lax.axis_index("s") * _NC + lax.axis_index("c")
    base = wid * rows_w

    pltpu.sync_copy(idx_hbm.at[pl.ds(base, rows_w)], idx_v)
    gather = pltpu.async_copy(x_hbm.at[idx_v], nbr_v, sem)
    pltpu.sync_copy(x_hbm.at[pl.ds(base, rows_w)], mine_v)
    pltpu.sync_copy(mine_v, out_hbm.at[pl.ds(base, rows_w)])
    pltpu.sync_copy(al_hbm.at[pl.ds(base, rows_w)], al_v)
    gather.wait()

    nchunk = d // _LANES

    def row_body(r, _):
        for c in range(nchunk):
            s = pl.ds(c * _LANES, _LANES)
            mine = mine_v[r, s]
            a = al_v[r, s]
            nbr_v[r, s] = mine + a * (nbr_v[r, s] - mine)
        return 0

    lax.fori_loop(0, rows_w, row_body, 0)
    pltpu.sync_copy(nbr_v, out_hbm.at[pl.ds(n + base, rows_w)])


def kernel(x):
    n, d = x.shape
    blk_r = 256
    nb_blocks = n // blk_r
    nw = _NC * _NS
    rows_w = n // nw

    key = jax.random.key(1)
    k1, k2 = jax.random.split(key)
    choice = jax.random.randint(k1, (1, n), 0, _N_NEIGHBORS)[0]
    alpha = jax.random.uniform(k2, (1, n, 1), dtype=x.dtype)[0]

    ranksel = (choice[:, None] == jnp.arange(8)[None, :]).astype(jnp.float32)
    alpha_full = jnp.broadcast_to(alpha, (n, d))

    xt = x.T

    nb_idx_f = pl.pallas_call(
        functools.partial(_knn_idx_kernel, blk_r=blk_r, n=n),
        grid=(nb_blocks,),
        in_specs=[
            pl.BlockSpec((n, d), lambda i: (0, 0)),
            pl.BlockSpec((d, n), lambda i: (0, 0)),
            pl.BlockSpec((blk_r, 8), lambda i: (i, 0)),
        ],
        out_specs=pl.BlockSpec((blk_r, 8), lambda i: (i, 0)),
        out_shape=jax.ShapeDtypeStruct((n, 8), jnp.float32),
    )(x, xt, ranksel)

    nb_idx = nb_idx_f[:, 0].astype(jnp.int32)

    sc = functools.partial(
        pl.kernel,
        out_type=jax.ShapeDtypeStruct((2 * n, d), jnp.float32),
        mesh=plsc.VectorSubcoreMesh(core_axis_name="c", subcore_axis_name="s"),
        scratch_types=[
            pltpu.VMEM((rows_w,), jnp.int32),
            pltpu.VMEM((rows_w, d), jnp.float32),
            pltpu.VMEM((rows_w, d), jnp.float32),
            pltpu.VMEM((rows_w, d), jnp.float32),
            pltpu.SemaphoreType.DMA,
        ],
    )(functools.partial(_sc_augment, n=n, d=d, rows_w=rows_w))

    return sc(x, nb_idx, alpha_full)


# speedup vs baseline: 34.0659x; 1.0495x over previous
import functools

import jax
import jax.numpy as jnp
from jax import lax
from jax.experimental import pallas as pl
from jax.experimental.pallas import tpu as pltpu
from jax.experimental.pallas import tpu_sc as plsc

_N_NEIGHBORS = 5
_BIG = 3.0e38

_NC = 2
_NS = 16
_LANES = 16


def _knn_idx_kernel(x_ref, xt_ref, ranksel_ref, out_ref, *, blk_r, n):
    i = pl.program_id(0)
    r0 = pl.multiple_of(i * blk_r, blk_r)
    xb = x_ref[pl.ds(r0, blk_r), :]

    dots = lax.dot_general(
        xb, xt_ref[:],
        (((1,), (0,)), ((), ())),
        precision=lax.Precision.DEFAULT,
        preferred_element_type=jnp.float32,
    )
    sq_cols = jnp.sum(xt_ref[:] * xt_ref[:], axis=0, keepdims=True)
    sq_rows = jnp.sum(xb * xb, axis=1, keepdims=True)
    d2 = sq_rows + sq_cols - 2.0 * dots

    cols_i = lax.broadcasted_iota(jnp.int32, (blk_r, n), 1)
    rows_i = i * blk_r + lax.broadcasted_iota(jnp.int32, (blk_r, n), 0)
    d2 = jnp.where(cols_i == rows_i, _BIG, d2)
    cols_f = cols_i.astype(jnp.float32)

    vstar = jnp.zeros((blk_r, 1), jnp.float32)
    m = jnp.min(d2, axis=1, keepdims=True)
    vstar = vstar + ranksel_ref[:, 0:1] * m
    for r in range(1, _N_NEIGHBORS):
        m = jnp.min(jnp.where(d2 > m, d2, _BIG), axis=1, keepdims=True)
        vstar = vstar + ranksel_ref[:, r:r + 1] * m
    nb_f = jnp.min(jnp.where(d2 == vstar, cols_f, _BIG), axis=1, keepdims=True)

    out_ref[0, 0, :] = jnp.reshape(nb_f.astype(jnp.int32), (blk_r,))


def _sc_augment(x_hbm, idx_hbm, al_hbm, out_hbm, idx_v, nbr_v, mine_v, al_v, sem,
                *, n, d, rows_w):
    wid = lax.axis_index("s") * _NC + lax.axis_index("c")
    base = wid * rows_w

    pltpu.sync_copy(idx_hbm.at[pl.ds(base, rows_w)], idx_v)
    gather = pltpu.async_copy(x_hbm.at[idx_v], nbr_v, sem)
    pltpu.sync_copy(x_hbm.at[pl.ds(base, rows_w)], mine_v)
    pltpu.sync_copy(mine_v, out_hbm.at[pl.ds(base, rows_w)])
    pltpu.sync_copy(al_hbm.at[pl.ds(base, rows_w)], al_v)
    gather.wait()

    nchunk = d // _LANES

    def row_body(r, _):
        for c in range(nchunk):
            s = pl.ds(c * _LANES, _LANES)
            mine = mine_v[r, s]
            a = al_v[r, s]
            nbr_v[r, s] = mine + a * (nbr_v[r, s] - mine)
        return 0

    lax.fori_loop(0, rows_w, row_body, 0)
    pltpu.sync_copy(nbr_v, out_hbm.at[pl.ds(n + base, rows_w)])


def kernel(x):
    n, d = x.shape
    blk_r = 512
    nb_blocks = n // blk_r
    nw = _NC * _NS
    rows_w = n // nw

    key = jax.random.key(1)
    k1, k2 = jax.random.split(key)
    choice = jax.random.randint(k1, (1, n), 0, _N_NEIGHBORS)[0]
    alpha = jax.random.uniform(k2, (1, n, 1), dtype=x.dtype)[0]

    ranksel = (choice[:, None] == jnp.arange(8)[None, :]).astype(jnp.float32)
    alpha_full = jnp.broadcast_to(alpha, (n, d))

    xt = x.T

    nb_idx_3d = pl.pallas_call(
        functools.partial(_knn_idx_kernel, blk_r=blk_r, n=n),
        grid=(nb_blocks,),
        in_specs=[
            pl.BlockSpec((n, d), lambda i: (0, 0)),
            pl.BlockSpec((d, n), lambda i: (0, 0)),
            pl.BlockSpec((blk_r, 8), lambda i: (i, 0)),
        ],
        out_specs=pl.BlockSpec((1, 1, blk_r), lambda i: (i, 0, 0)),
        out_shape=jax.ShapeDtypeStruct((nb_blocks, 1, blk_r), jnp.int32),
    )(x, xt, ranksel)

    nb_idx = nb_idx_3d.reshape(n)

    sc = functools.partial(
        pl.kernel,
        out_type=jax.ShapeDtypeStruct((2 * n, d), jnp.float32),
        mesh=plsc.VectorSubcoreMesh(core_axis_name="c", subcore_axis_name="s"),
        scratch_types=[
            pltpu.VMEM((rows_w,), jnp.int32),
            pltpu.VMEM((rows_w, d), jnp.float32),
            pltpu.VMEM((rows_w, d), jnp.float32),
            pltpu.VMEM((rows_w, d), jnp.float32),
            pltpu.SemaphoreType.DMA,
        ],
    )(functools.partial(_sc_augment, n=n, d=d, rows_w=rows_w))

    return sc(x, nb_idx, alpha_full)


# speedup vs baseline: 34.8479x; 1.0230x over previous
import functools

import jax
import jax.numpy as jnp
from jax import lax
from jax.experimental import pallas as pl
from jax.experimental.pallas import tpu as pltpu
from jax.experimental.pallas import tpu_sc as plsc

_N_NEIGHBORS = 5
_BIG = 3.0e38

_NC = 2
_NS = 16
_LANES = 16


def _knn_idx_kernel(x_ref, xtm_ref, ranksel_ref, out_ref, sq_ref, *, blk_r, n):
    i = pl.program_id(0)
    r0 = pl.multiple_of(i * blk_r, blk_r)
    xb = x_ref[pl.ds(r0, blk_r), :]

    @pl.when(i == 0)
    def _():
        sq_ref[:, :] = jnp.sum(xtm_ref[:] * xtm_ref[:], axis=0, keepdims=True) * 0.25

    dots = lax.dot_general(
        xb, xtm_ref[:],
        (((1,), (0,)), ((), ())),
        precision=lax.Precision.DEFAULT,
        preferred_element_type=jnp.float32,
    )
    sq_rows = jnp.sum(xb * xb, axis=1, keepdims=True)
    d2 = (sq_rows + sq_ref[:, :]) + dots

    cols_i = lax.broadcasted_iota(jnp.int32, (blk_r, n), 1)
    rows_i = i * blk_r + lax.broadcasted_iota(jnp.int32, (blk_r, n), 0)
    d2 = jnp.where(cols_i == rows_i, _BIG, d2)
    cols_f = cols_i.astype(jnp.float32)

    vstar = jnp.zeros((blk_r, 1), jnp.float32)
    m = jnp.min(d2, axis=1, keepdims=True)
    vstar = vstar + ranksel_ref[:, 0:1] * m
    for r in range(1, _N_NEIGHBORS):
        m = jnp.min(jnp.where(d2 > m, d2, _BIG), axis=1, keepdims=True)
        vstar = vstar + ranksel_ref[:, r:r + 1] * m
    nb_f = jnp.min(jnp.where(d2 == vstar, cols_f, _BIG), axis=1, keepdims=True)

    out_ref[0, 0, :] = jnp.reshape(nb_f.astype(jnp.int32), (blk_r,))


def _sc_augment(x_hbm, idx_hbm, al_hbm, out_hbm, idx_v, nbr_v, mine_v, al_v, sem,
                *, n, d, rows_w):
    wid = lax.axis_index("s") * _NC + lax.axis_index("c")
    base = wid * rows_w

    pltpu.sync_copy(idx_hbm.at[pl.ds(base, rows_w)], idx_v)
    gather = pltpu.async_copy(x_hbm.at[idx_v], nbr_v, sem)
    pltpu.sync_copy(x_hbm.at[pl.ds(base, rows_w)], mine_v)
    pltpu.sync_copy(mine_v, out_hbm.at[pl.ds(base, rows_w)])
    pltpu.sync_copy(al_hbm.at[pl.ds(base, rows_w)], al_v)
    gather.wait()

    nchunk = d // _LANES

    def row_body(r, _):
        for c in range(nchunk):
            s = pl.ds(c * _LANES, _LANES)
            mine = mine_v[r, s]
            a = al_v[r, s]
            nbr_v[r, s] = mine + a * (nbr_v[r, s] - mine)
        return 0

    lax.fori_loop(0, rows_w, row_body, 0)
    pltpu.sync_copy(nbr_v, out_hbm.at[pl.ds(n + base, rows_w)])


def kernel(x):
    n, d = x.shape
    blk_r = 512
    nb_blocks = n // blk_r
    nw = _NC * _NS
    rows_w = n // nw

    key = jax.random.key(1)
    k1, k2 = jax.random.split(key)
    choice = jax.random.randint(k1, (1, n), 0, _N_NEIGHBORS)[0]
    alpha = jax.random.uniform(k2, (1, n, 1), dtype=x.dtype)[0]

    ranksel = (choice[:, None] == jnp.arange(8)[None, :]).astype(jnp.float32)
    alpha_full = jnp.broadcast_to(alpha, (n, d))

    xtm = -2.0 * x.T

    nb_idx_3d = pl.pallas_call(
        functools.partial(_knn_idx_kernel, blk_r=blk_r, n=n),
        grid=(nb_blocks,),
        in_specs=[
            pl.BlockSpec((n, d), lambda i: (0, 0)),
            pl.BlockSpec((d, n), lambda i: (0, 0)),
            pl.BlockSpec((blk_r, 8), lambda i: (i, 0)),
        ],
        out_specs=pl.BlockSpec((1, 1, blk_r), lambda i: (i, 0, 0)),
        out_shape=jax.ShapeDtypeStruct((nb_blocks, 1, blk_r), jnp.int32),
        scratch_shapes=[pltpu.VMEM((1, n), jnp.float32)],
    )(x, xtm, ranksel)

    nb_idx = nb_idx_3d.reshape(n)

    sc = functools.partial(
        pl.kernel,
        out_type=jax.ShapeDtypeStruct((2 * n, d), jnp.float32),
        mesh=plsc.VectorSubcoreMesh(core_axis_name="c", subcore_axis_name="s"),
        scratch_types=[
            pltpu.VMEM((rows_w,), jnp.int32),
            pltpu.VMEM((rows_w, d), jnp.float32),
            pltpu.VMEM((rows_w, d), jnp.float32),
            pltpu.VMEM((rows_w, d), jnp.float32),
            pltpu.SemaphoreType.DMA,
        ],
    )(functools.partial(_sc_augment, n=n, d=d, rows_w=rows_w))

    return sc(x, nb_idx, alpha_full)


# speedup vs baseline: 35.4109x; 1.0162x over previous
import functools

import jax
import jax.numpy as jnp
from jax import lax
from jax.experimental import pallas as pl
from jax.experimental.pallas import tpu as pltpu
from jax.experimental.pallas import tpu_sc as plsc

_N_NEIGHBORS = 5
_BIG = 3.0e38

_NC = 2
_NS = 16
_LANES = 16


def _knn_idx_kernel(x_ref, xtm_ref, ranksel_ref, out_ref, sq_ref, *, blk_r, n):
    i = pl.program_id(0)
    r0 = pl.multiple_of(i * blk_r, blk_r)
    xb = x_ref[pl.ds(r0, blk_r), :]

    @pl.when(i == 0)
    def _():
        sq_ref[:, :] = jnp.sum(xtm_ref[:] * xtm_ref[:], axis=0, keepdims=True) * 0.25

    dots = lax.dot_general(
        xb, xtm_ref[:],
        (((1,), (0,)), ((), ())),
        precision=lax.Precision.DEFAULT,
        preferred_element_type=jnp.float32,
    )
    sq_rows = jnp.sum(xb * xb, axis=1, keepdims=True)
    d2 = (sq_rows + sq_ref[:, :]) + dots

    cols_i = lax.broadcasted_iota(jnp.int32, (blk_r, n), 1)
    rows_i = i * blk_r + lax.broadcasted_iota(jnp.int32, (blk_r, n), 0)
    d2 = jnp.where(cols_i == rows_i, _BIG, d2)
    cols_f = cols_i.astype(jnp.float32)

    vstar = jnp.zeros((blk_r, 1), jnp.float32)
    m = jnp.min(d2, axis=1, keepdims=True)
    vstar = vstar + ranksel_ref[:, 0:1] * m
    for r in range(1, _N_NEIGHBORS):
        m = jnp.min(jnp.where(d2 > m, d2, _BIG), axis=1, keepdims=True)
        vstar = vstar + ranksel_ref[:, r:r + 1] * m
    nb_f = jnp.min(jnp.where(d2 == vstar, cols_f, _BIG), axis=1, keepdims=True)

    out_ref[0, 0, :] = jnp.reshape(nb_f.astype(jnp.int32), (blk_r,))


def _sc_augment(x_hbm, idx_hbm, al_hbm, out_hbm, idx_v, nbr_v, mine_v, al_v, sem,
                *, n, d, rows_w):
    wid = lax.axis_index("s") * _NC + lax.axis_index("c")
    base = wid * rows_w

    pltpu.sync_copy(idx_hbm.at[pl.ds(base, rows_w)], idx_v)
    gather = pltpu.async_copy(x_hbm.at[idx_v], nbr_v, sem)
    pltpu.sync_copy(x_hbm.at[pl.ds(base, rows_w)], mine_v)
    pltpu.sync_copy(mine_v, out_hbm.at[pl.ds(base, rows_w)])
    pltpu.sync_copy(al_hbm.at[pl.ds(base, rows_w)], al_v)
    gather.wait()

    nchunk = d // _LANES

    def row_body(r, _):
        for c in range(nchunk):
            s = pl.ds(c * _LANES, _LANES)
            mine = mine_v[r, s]
            a = al_v[r, s]
            nbr_v[r, s] = mine + a * (nbr_v[r, s] - mine)
        return 0

    lax.fori_loop(0, rows_w, row_body, 0)
    pltpu.sync_copy(nbr_v, out_hbm.at[pl.ds(n + base, rows_w)])


def kernel(x):
    n, d = x.shape
    blk_r = 1024
    nb_blocks = n // blk_r
    nw = _NC * _NS
    rows_w = n // nw

    key = jax.random.key(1)
    k1, k2 = jax.random.split(key)
    choice = jax.random.randint(k1, (1, n), 0, _N_NEIGHBORS)[0]
    alpha = jax.random.uniform(k2, (1, n, 1), dtype=x.dtype)[0]

    ranksel = (choice[:, None] == jnp.arange(8)[None, :]).astype(jnp.float32)
    alpha_full = jnp.broadcast_to(alpha, (n, d))

    xtm = -2.0 * x.T

    nb_idx_3d = pl.pallas_call(
        functools.partial(_knn_idx_kernel, blk_r=blk_r, n=n),
        grid=(nb_blocks,),
        in_specs=[
            pl.BlockSpec((n, d), lambda i: (0, 0)),
            pl.BlockSpec((d, n), lambda i: (0, 0)),
            pl.BlockSpec((blk_r, 8), lambda i: (i, 0)),
        ],
        out_specs=pl.BlockSpec((1, 1, blk_r), lambda i: (i, 0, 0)),
        out_shape=jax.ShapeDtypeStruct((nb_blocks, 1, blk_r), jnp.int32),
        scratch_shapes=[pltpu.VMEM((1, n), jnp.float32)],
    )(x, xtm, ranksel)

    nb_idx = nb_idx_3d.reshape(n)

    sc = functools.partial(
        pl.kernel,
        out_type=jax.ShapeDtypeStruct((2 * n, d), jnp.float32),
        mesh=plsc.VectorSubcoreMesh(core_axis_name="c", subcore_axis_name="s"),
        scratch_types=[
            pltpu.VMEM((rows_w,), jnp.int32),
            pltpu.VMEM((rows_w, d), jnp.float32),
            pltpu.VMEM((rows_w, d), jnp.float32),
            pltpu.VMEM((rows_w, d), jnp.float32),
            pltpu.SemaphoreType.DMA,
        ],
    )(functools.partial(_sc_augment, n=n, d=d, rows_w=rows_w))

    return sc(x, nb_idx, alpha_full)


# speedup vs baseline: 35.7387x; 1.0093x over previous
import functools

import jax
import jax.numpy as jnp
from jax import lax
from jax.experimental import pallas as pl
from jax.experimental.pallas import tpu as pltpu
from jax.experimental.pallas import tpu_sc as plsc

_N_NEIGHBORS = 5
_BIG = 3.0e38

_NC = 2
_NS = 16
_LANES = 16


def _knn_idx_kernel(x_ref, xtm_ref, ranksel_ref, out_ref, sq_ref, *, blk_r, n):
    i = pl.program_id(0)
    r0 = pl.multiple_of(i * blk_r, blk_r)
    xb = x_ref[pl.ds(r0, blk_r), :]

    @pl.when(i == 0)
    def _():
        sq_ref[:, :] = jnp.sum(xtm_ref[:] * xtm_ref[:], axis=0, keepdims=True) * 0.25

    dots = lax.dot_general(
        xb, xtm_ref[:],
        (((1,), (0,)), ((), ())),
        precision=lax.Precision.DEFAULT,
        preferred_element_type=jnp.float32,
    )
    sq_rows = jnp.sum(xb * xb, axis=1, keepdims=True)
    d2 = (sq_rows + sq_ref[:, :]) + dots

    cols_i = lax.broadcasted_iota(jnp.int32, (blk_r, n), 1)
    rows_i = i * blk_r + lax.broadcasted_iota(jnp.int32, (blk_r, n), 0)
    d2 = jnp.where(cols_i == rows_i, _BIG, d2)
    cols_f = cols_i.astype(jnp.float32)

    vstar = jnp.zeros((blk_r, 1), jnp.float32)
    m = jnp.min(d2, axis=1, keepdims=True)
    vstar = vstar + ranksel_ref[:, 0:1] * m
    for r in range(1, _N_NEIGHBORS):
        m = jnp.min(jnp.where(d2 > m, d2, _BIG), axis=1, keepdims=True)
        vstar = vstar + ranksel_ref[:, r:r + 1] * m
    nb_f = jnp.min(jnp.where(d2 == vstar, cols_f, _BIG), axis=1, keepdims=True)

    out_ref[0, 0, :] = jnp.reshape(nb_f.astype(jnp.int32), (blk_r,))


def _sc_augment(x_hbm, idx_hbm, al_hbm, out_hbm, idx_v, nbr_v, mine_v, al_v, sem,
                *, n, d, rows_w):
    wid = lax.axis_index("s") * _NC + lax.axis_index("c")
    base = wid * rows_w

    pltpu.sync_copy(idx_hbm.at[pl.ds(base, rows_w)], idx_v)
    gather = pltpu.async_copy(x_hbm.at[idx_v], nbr_v, sem)
    pltpu.sync_copy(x_hbm.at[pl.ds(base, rows_w)], mine_v)
    pltpu.sync_copy(mine_v, out_hbm.at[pl.ds(base, rows_w)])
    pltpu.sync_copy(al_hbm.at[pl.ds(base, rows_w)], al_v)
    gather.wait()

    nchunk = d // _LANES

    def row_body(r, _):
        for c in range(nchunk):
            s = pl.ds(c * _LANES, _LANES)
            mine = mine_v[r, s]
            a = al_v[r, s]
            nbr_v[r, s] = mine + a * (nbr_v[r, s] - mine)
        return 0

    lax.fori_loop(0, rows_w, row_body, 0)
    pltpu.sync_copy(nbr_v, out_hbm.at[pl.ds(n + base, rows_w)])


def kernel(x):
    n, d = x.shape
    blk_r = 2048
    nb_blocks = n // blk_r
    nw = _NC * _NS
    rows_w = n // nw

    key = jax.random.key(1)
    k1, k2 = jax.random.split(key)
    choice = jax.random.randint(k1, (1, n), 0, _N_NEIGHBORS)[0]
    alpha = jax.random.uniform(k2, (1, n, 1), dtype=x.dtype)[0]

    ranksel = (choice[:, None] == jnp.arange(8)[None, :]).astype(jnp.float32)
    alpha_full = jnp.broadcast_to(alpha, (n, d))

    xtm = -2.0 * x.T

    nb_idx_3d = pl.pallas_call(
        functools.partial(_knn_idx_kernel, blk_r=blk_r, n=n),
        grid=(nb_blocks,),
        in_specs=[
            pl.BlockSpec((n, d), lambda i: (0, 0)),
            pl.BlockSpec((d, n), lambda i: (0, 0)),
            pl.BlockSpec((blk_r, 8), lambda i: (i, 0)),
        ],
        out_specs=pl.BlockSpec((1, 1, blk_r), lambda i: (i, 0, 0)),
        out_shape=jax.ShapeDtypeStruct((nb_blocks, 1, blk_r), jnp.int32),
        scratch_shapes=[pltpu.VMEM((1, n), jnp.float32)],
    )(x, xtm, ranksel)

    nb_idx = nb_idx_3d.reshape(n)

    sc = functools.partial(
        pl.kernel,
        out_type=jax.ShapeDtypeStruct((2 * n, d), jnp.float32),
        mesh=plsc.VectorSubcoreMesh(core_axis_name="c", subcore_axis_name="s"),
        scratch_types=[
            pltpu.VMEM((rows_w,), jnp.int32),
            pltpu.VMEM((rows_w, d), jnp.float32),
            pltpu.VMEM((rows_w, d), jnp.float32),
            pltpu.VMEM((rows_w, d), jnp.float32),
            pltpu.SemaphoreType.DMA,
        ],
    )(functools.partial(_sc_augment, n=n, d=d, rows_w=rows_w))

    return sc(x, nb_idx, alpha_full)
